# baseline (speedup 1.0000x reference)
import functools

import jax
import jax.numpy as jnp
from jax import lax
from jax.experimental import pallas as pl
from jax.experimental.pallas import tpu as pltpu
from jax.experimental.pallas import tpu_sc as plsc

B = 16384
F = 16
NC = 2
NS = 16
L = 16
NW = NC * NS
BPW = B // NW
BLOCKS = BPW // L

_mesh = plsc.VectorSubcoreMesh(core_axis_name="c", subcore_axis_name="s")


@functools.partial(
    pl.kernel,
    out_type=jax.ShapeDtypeStruct((B,), jnp.float32),
    mesh=_mesh,
    scratch_types=[
        pltpu.VMEM((BPW,), jnp.int32),
        pltpu.VMEM((BPW,), jnp.int32),
        pltpu.VMEM((BPW, F), jnp.float32),
        pltpu.VMEM((BPW, F), jnp.float32),
        pltpu.VMEM((BPW, F), jnp.float32),
        pltpu.VMEM((BPW,), jnp.float32),
        pltpu.VMEM((BPW,), jnp.float32),
        pltpu.VMEM((BPW,), jnp.float32),
        pltpu.SemaphoreType.DMA,
    ],
    compiler_params=pltpu.CompilerParams(
        needs_layout_passes=False, use_tc_tiling_on_sc=False),
)
def _svdpp_sc(user_h, item_h, uf_h, if_h, ub_h, ib_h, uimp_h, out_h,
              idx_u, idx_i, uf_v, ui_v, if_v, ub_v, ib_v, out_v, sem):
    wid = lax.axis_index("s") * NC + lax.axis_index("c")
    base = wid * BPW

    pltpu.sync_copy(user_h.at[pl.ds(base, BPW)], idx_u)
    pltpu.sync_copy(item_h.at[pl.ds(base, BPW)], idx_i)

    c1 = pltpu.async_copy(uf_h.at[idx_u], uf_v, sem)
    c2 = pltpu.async_copy(uimp_h.at[idx_u], ui_v, sem)
    c3 = pltpu.async_copy(if_h.at[idx_i], if_v, sem)
    c4 = pltpu.async_copy(ub_h.at[idx_u], ub_v, sem)
    c5 = pltpu.async_copy(ib_h.at[idx_i], ib_v, sem)
    c1.wait()
    c2.wait()
    c3.wait()
    c4.wait()
    c5.wait()

    iota = lax.iota(jnp.int32, L)

    def block(blk, carry):
        rows = blk * L + iota
        acc = jnp.zeros((L,), jnp.float32)
        for j in range(F):
            colj = jnp.full((L,), j, jnp.int32)
            a = plsc.load_gather(uf_v, [rows, colj])
            b = plsc.load_gather(ui_v, [rows, colj])
            c = plsc.load_gather(if_v, [rows, colj])
            acc = acc + (a + b) * c
        o = blk * L
        pred = acc + ub_v[pl.ds(o, L)] + ib_v[pl.ds(o, L)]
        out_v[pl.ds(o, L)] = 1.0 / (1.0 + jnp.exp(-pred))
        return carry

    lax.fori_loop(0, BLOCKS, block, 0)

    pltpu.sync_copy(out_v, out_h.at[pl.ds(base, BPW)])


def kernel(user, item, user_factors, item_factors, user_biases,
           item_biases, user_implicit):
    ub = user_biases.reshape((-1,))
    ib = item_biases.reshape((-1,))
    return _svdpp_sc(user, item, user_factors, item_factors, ub, ib,
                     user_implicit)
